# Initial kernel scaffold; baseline (speedup 1.0000x reference)
import jax
import jax.numpy as jnp
from jax.experimental import pallas as pl


def kernel(U, V, edge_index, batch):
    raise NotImplementedError("write your pallas kernel here")


# speedup vs baseline: 12.5420x; 12.5420x over previous
import functools

import jax
import jax.numpy as jnp
from jax import lax
from jax.experimental import pallas as pl
from jax.experimental.pallas import tpu as pltpu
from jax.experimental.pallas import tpu_sc as plsc

NNODES = 10000
RANK = 16
NEDGES = 320000
NBATCH = 8192
EMB = 128

NC = 2
NS = 16
NW = NC * NS

GRP = 128
EPT = 10240
NGRP = EPT // GRP
EPAD = EPT * NW
DUMMY = NNODES
NROWS = 10112
RPT = NROWS // NS

BPT = NBATCH // NW
BGRP = BPT // GRP

_MESH = plsc.VectorSubcoreMesh(
    core_axis_name="c", subcore_axis_name="s", num_cores=NC, num_subcores=NS
)


def _scatter_body(u_hbm, srcg, dstg, zeros_hbm, ones_hbm,
                  acc0, deg0, acc1, deg1,
                  acc_sh, deg_sh, sidx, didx, rows, ones_v, sem):
    cid = lax.axis_index("c")
    sid = lax.axis_index("s")
    wid = cid * NS + sid
    r0 = sid * RPT

    pltpu.sync_copy(zeros_hbm.at[pl.ds(r0, RPT)], acc_sh.at[pl.ds(r0, RPT)])
    pltpu.sync_copy(zeros_hbm.at[pl.ds(r0, RPT)], deg_sh.at[pl.ds(r0, RPT)])

    pltpu.sync_copy(srcg.at[pl.ds(wid * NGRP, NGRP)], sidx)
    pltpu.sync_copy(dstg.at[pl.ds(wid * NGRP, NGRP)], didx)
    pltpu.sync_copy(ones_hbm, ones_v)

    plsc.subcore_barrier()

    def body(j, carry):
        pltpu.async_copy(u_hbm.at[sidx.at[j]], rows, sem).wait()
        pltpu.sync_copy(rows, acc_sh.at[didx.at[j]], add=True)
        pltpu.sync_copy(ones_v, deg_sh.at[didx.at[j]], add=True)
        return carry

    lax.fori_loop(0, NGRP, body, 0)

    plsc.subcore_barrier()

    @pl.when(cid == 0)
    def _():
        pltpu.sync_copy(acc_sh.at[pl.ds(r0, RPT)], acc0.at[pl.ds(r0, RPT)])
        pltpu.sync_copy(deg_sh.at[pl.ds(r0, RPT)], deg0.at[pl.ds(r0, RPT)])

    @pl.when(cid == 1)
    def _():
        pltpu.sync_copy(acc_sh.at[pl.ds(r0, RPT)], acc1.at[pl.ds(r0, RPT)])
        pltpu.sync_copy(deg_sh.at[pl.ds(r0, RPT)], deg1.at[pl.ds(r0, RPT)])


_SC_PARAMS = pltpu.CompilerParams(use_tc_tiling_on_sc=False)

_scatter_kernel = functools.partial(
    pl.kernel,
    out_type=[jax.ShapeDtypeStruct((NROWS, RANK), jnp.float32)] * 4,
    mesh=_MESH,
    compiler_params=_SC_PARAMS,
    scratch_types=[
        pltpu.VMEM_SHARED((NROWS, RANK), jnp.float32),
        pltpu.VMEM_SHARED((NROWS, RANK), jnp.float32),
        pltpu.VMEM((NGRP, GRP), jnp.int32),
        pltpu.VMEM((NGRP, GRP), jnp.int32),
        pltpu.VMEM((GRP, RANK), jnp.float32),
        pltpu.VMEM((GRP, RANK), jnp.float32),
        pltpu.SemaphoreType.DMA,
    ],
)(_scatter_body)


def _mean_gather_body(acc0, deg0, acc1, deg1, batchg, outr,
                      bidx, a0, a1, d0, d1, ov, sem):
    cid = lax.axis_index("c")
    sid = lax.axis_index("s")
    wid = cid * NS + sid

    pltpu.sync_copy(batchg.at[pl.ds(wid * BGRP, BGRP)], bidx)
    for j in range(BGRP):
        sl = pl.ds(j * GRP, GRP)
        pltpu.async_copy(acc0.at[bidx.at[j]], a0.at[sl], sem).wait()
        pltpu.async_copy(acc1.at[bidx.at[j]], a1.at[sl], sem).wait()
        pltpu.async_copy(deg0.at[bidx.at[j]], d0.at[sl], sem).wait()
        pltpu.async_copy(deg1.at[bidx.at[j]], d1.at[sl], sem).wait()

    def body(i, carry):
        a = a0[i] + a1[i]
        d = d0[i] + d1[i]
        ov[i] = a / jnp.maximum(d, 1.0)
        return carry

    lax.fori_loop(0, BPT, body, 0)

    pltpu.sync_copy(ov, outr.at[pl.ds(wid * BPT, BPT)])


_mean_gather_kernel = functools.partial(
    pl.kernel,
    out_type=jax.ShapeDtypeStruct((NBATCH, RANK), jnp.float32),
    mesh=_MESH,
    compiler_params=_SC_PARAMS,
    scratch_types=[
        pltpu.VMEM((BGRP, GRP), jnp.int32),
        pltpu.VMEM((BPT, RANK), jnp.float32),
        pltpu.VMEM((BPT, RANK), jnp.float32),
        pltpu.VMEM((BPT, RANK), jnp.float32),
        pltpu.VMEM((BPT, RANK), jnp.float32),
        pltpu.VMEM((BPT, RANK), jnp.float32),
        pltpu.SemaphoreType.DMA,
    ],
)(_mean_gather_body)


def _mm_body(x_ref, v_ref, o_ref):
    o_ref[...] = jnp.dot(x_ref[...], v_ref[...],
                         preferred_element_type=jnp.float32)


def _expand(x, v):
    blk = 1024
    return pl.pallas_call(
        _mm_body,
        grid=(NBATCH // blk,),
        in_specs=[
            pl.BlockSpec((blk, RANK), lambda i: (i, 0)),
            pl.BlockSpec((RANK, EMB), lambda i: (0, 0)),
        ],
        out_specs=pl.BlockSpec((blk, EMB), lambda i: (i, 0)),
        out_shape=jax.ShapeDtypeStruct((NBATCH, EMB), jnp.float32),
    )(x, v)


def kernel(U, V, edge_index, batch):
    pad = EPAD - NEDGES
    src = jnp.concatenate([edge_index[0], jnp.zeros((pad,), jnp.int32)])
    dst = jnp.concatenate(
        [edge_index[1], jnp.full((pad,), DUMMY, jnp.int32)])
    srcg = src.reshape(EPAD // GRP, GRP)
    dstg = dst.reshape(EPAD // GRP, GRP)
    zeros = jnp.zeros((NROWS, RANK), jnp.float32)
    ones = jnp.ones((GRP, RANK), jnp.float32)

    acc0, deg0, acc1, deg1 = _scatter_kernel(U, srcg, dstg, zeros, ones)
    outr = _mean_gather_kernel(acc0, deg0, acc1, deg1,
                               batch.reshape(NBATCH // GRP, GRP))
    return _expand(outr, V)


# speedup vs baseline: 16.6128x; 1.3246x over previous
import functools

import jax
import jax.numpy as jnp
from jax import lax
from jax.experimental import pallas as pl
from jax.experimental.pallas import tpu as pltpu
from jax.experimental.pallas import tpu_sc as plsc

NNODES = 10000
RANK = 16
NEDGES = 320000
NBATCH = 8192
EMB = 128

NC = 2
NS = 16
NW = NC * NS

GRP = 128
EPT = 10240
NGRP = EPT // GRP
EPAD = EPT * NW
DUMMY = NNODES
NROWS = 10112
RPT = NROWS // NS

BPT = NBATCH // NW
BGRP = BPT // GRP

_MESH = plsc.VectorSubcoreMesh(
    core_axis_name="c", subcore_axis_name="s", num_cores=NC, num_subcores=NS
)


NBUF = 8
NCHUNK = NGRP // NBUF


def _scatter_body(u_hbm, srcg, dstg, zeros_hbm, ones_hbm,
                  acc0, deg0, acc1, deg1,
                  acc_sh, deg_sh, sidx, didx, rows, ones_v,
                  gsem, ssem, dsem):
    cid = lax.axis_index("c")
    sid = lax.axis_index("s")
    wid = cid * NS + sid
    r0 = sid * RPT

    pltpu.sync_copy(zeros_hbm.at[pl.ds(r0, RPT)], acc_sh.at[pl.ds(r0, RPT)])
    pltpu.sync_copy(zeros_hbm.at[pl.ds(r0, RPT)], deg_sh.at[pl.ds(r0, RPT)])

    pltpu.sync_copy(srcg.at[pl.ds(wid * NGRP, NGRP)], sidx)
    pltpu.sync_copy(dstg.at[pl.ds(wid * NGRP, NGRP)], didx)
    pltpu.sync_copy(ones_hbm, ones_v)

    plsc.subcore_barrier()

    def body(c, carry):
        j0 = c * NBUF
        gds = [pltpu.async_copy(u_hbm.at[sidx.at[j0 + b]], rows.at[b], gsem)
               for b in range(NBUF)]
        dds = [pltpu.async_copy(ones_v, deg_sh.at[didx.at[j0 + b]], dsem,
                                add=True)
               for b in range(NBUF)]
        for d in gds:
            d.wait()
        sds = [pltpu.async_copy(rows.at[b], acc_sh.at[didx.at[j0 + b]], ssem,
                                add=True)
               for b in range(NBUF)]
        for d in sds:
            d.wait()
        for d in dds:
            d.wait()
        return carry

    lax.fori_loop(0, NCHUNK, body, 0)

    plsc.subcore_barrier()

    @pl.when(cid == 0)
    def _():
        pltpu.sync_copy(acc_sh.at[pl.ds(r0, RPT)], acc0.at[pl.ds(r0, RPT)])
        pltpu.sync_copy(deg_sh.at[pl.ds(r0, RPT)], deg0.at[pl.ds(r0, RPT)])

    @pl.when(cid == 1)
    def _():
        pltpu.sync_copy(acc_sh.at[pl.ds(r0, RPT)], acc1.at[pl.ds(r0, RPT)])
        pltpu.sync_copy(deg_sh.at[pl.ds(r0, RPT)], deg1.at[pl.ds(r0, RPT)])


_SC_PARAMS = pltpu.CompilerParams(use_tc_tiling_on_sc=False)

_scatter_kernel = functools.partial(
    pl.kernel,
    out_type=[jax.ShapeDtypeStruct((NROWS, RANK), jnp.float32)] * 4,
    mesh=_MESH,
    compiler_params=_SC_PARAMS,
    scratch_types=[
        pltpu.VMEM_SHARED((NROWS, RANK), jnp.float32),
        pltpu.VMEM_SHARED((NROWS, RANK), jnp.float32),
        pltpu.VMEM((NGRP, GRP), jnp.int32),
        pltpu.VMEM((NGRP, GRP), jnp.int32),
        pltpu.VMEM((NBUF, GRP, RANK), jnp.float32),
        pltpu.VMEM((GRP, RANK), jnp.float32),
        pltpu.SemaphoreType.DMA,
        pltpu.SemaphoreType.DMA,
        pltpu.SemaphoreType.DMA,
    ],
)(_scatter_body)


def _mean_gather_body(acc0, deg0, acc1, deg1, batchg, outr,
                      bidx, a0, a1, d0, d1, ov, sem):
    cid = lax.axis_index("c")
    sid = lax.axis_index("s")
    wid = cid * NS + sid

    pltpu.sync_copy(batchg.at[pl.ds(wid * BGRP, BGRP)], bidx)
    ds = []
    for j in range(BGRP):
        sl = pl.ds(j * GRP, GRP)
        ds.append(pltpu.async_copy(acc0.at[bidx.at[j]], a0.at[sl], sem))
        ds.append(pltpu.async_copy(acc1.at[bidx.at[j]], a1.at[sl], sem))
        ds.append(pltpu.async_copy(deg0.at[bidx.at[j]], d0.at[sl], sem))
        ds.append(pltpu.async_copy(deg1.at[bidx.at[j]], d1.at[sl], sem))
    for d in ds:
        d.wait()

    def body(i, carry):
        a = a0[i] + a1[i]
        d = d0[i] + d1[i]
        ov[i] = a / jnp.maximum(d, 1.0)
        return carry

    lax.fori_loop(0, BPT, body, 0)

    pltpu.sync_copy(ov, outr.at[pl.ds(wid * BPT, BPT)])


_mean_gather_kernel = functools.partial(
    pl.kernel,
    out_type=jax.ShapeDtypeStruct((NBATCH, RANK), jnp.float32),
    mesh=_MESH,
    compiler_params=_SC_PARAMS,
    scratch_types=[
        pltpu.VMEM((BGRP, GRP), jnp.int32),
        pltpu.VMEM((BPT, RANK), jnp.float32),
        pltpu.VMEM((BPT, RANK), jnp.float32),
        pltpu.VMEM((BPT, RANK), jnp.float32),
        pltpu.VMEM((BPT, RANK), jnp.float32),
        pltpu.VMEM((BPT, RANK), jnp.float32),
        pltpu.SemaphoreType.DMA,
    ],
)(_mean_gather_body)


def _mm_body(x_ref, v_ref, o_ref):
    o_ref[...] = jnp.dot(x_ref[...], v_ref[...],
                         preferred_element_type=jnp.float32)


def _expand(x, v):
    blk = 1024
    return pl.pallas_call(
        _mm_body,
        grid=(NBATCH // blk,),
        in_specs=[
            pl.BlockSpec((blk, RANK), lambda i: (i, 0)),
            pl.BlockSpec((RANK, EMB), lambda i: (0, 0)),
        ],
        out_specs=pl.BlockSpec((blk, EMB), lambda i: (i, 0)),
        out_shape=jax.ShapeDtypeStruct((NBATCH, EMB), jnp.float32),
    )(x, v)


def kernel(U, V, edge_index, batch):
    pad = EPAD - NEDGES
    src = jnp.concatenate([edge_index[0], jnp.zeros((pad,), jnp.int32)])
    dst = jnp.concatenate(
        [edge_index[1], jnp.full((pad,), DUMMY, jnp.int32)])
    srcg = src.reshape(EPAD // GRP, GRP)
    dstg = dst.reshape(EPAD // GRP, GRP)
    zeros = jnp.zeros((NROWS, RANK), jnp.float32)
    ones = jnp.ones((GRP, RANK), jnp.float32)

    acc0, deg0, acc1, deg1 = _scatter_kernel(U, srcg, dstg, zeros, ones)
    outr = _mean_gather_kernel(acc0, deg0, acc1, deg1,
                               batch.reshape(NBATCH // GRP, GRP))
    return _expand(outr, V)
